# baseline (speedup 1.0000x reference)
import jax
import jax.numpy as jnp
from jax import lax
from jax.experimental import pallas as pl
from jax.experimental.pallas import tpu as pltpu
from jax.experimental.pallas import tpu_sc as plsc

N = 10000
E = 320000
DIM = 128

NC = 2
NS = 16
NW = NC * NS
K = 128
ROWS_PER_W = 80
EPAD = NW * ROWS_PER_W * K
NPAD = 10240
STRIPE = NPAD // NS
DUMP = NPAD - 1


def _mesh():
    return plsc.VectorSubcoreMesh(core_axis_name="c", subcore_axis_name="s")


def _deg_body(edges_hbm, out_hbm, idx_v, ones_v, zero_v, deg_sh, sem):
    c = lax.axis_index("c")
    s = lax.axis_index("s")
    w = s * NC + c
    pltpu.sync_copy(edges_hbm.at[w], idx_v)
    for i in range(K // 16):
        ones_v[pl.ds(i * 16, 16)] = jnp.ones((16,), jnp.float32)

    def zb(t, carry):
        zero_v[pl.ds(t * 16, 16)] = jnp.zeros((16,), jnp.float32)
        return carry

    lax.fori_loop(0, STRIPE // 16, zb, 0)
    pltpu.sync_copy(zero_v, deg_sh.at[pl.ds(s * STRIPE, STRIPE)])
    plsc.subcore_barrier()

    def step(j, carry):
        pltpu.sync_copy(ones_v, deg_sh.at[idx_v.at[j, 1]], add=True)
        return carry

    lax.fori_loop(0, ROWS_PER_W, step, 0)
    plsc.subcore_barrier()
    pltpu.sync_copy(deg_sh.at[pl.ds(s * STRIPE, STRIPE)],
                    out_hbm.at[c, 0, pl.ds(s * STRIPE, STRIPE)])


def _sc_degree(edges4d):
    kern = pl.kernel(
        _deg_body,
        out_type=jax.ShapeDtypeStruct((NC, 1, NPAD), jnp.float32),
        mesh=_mesh(),
        scratch_types=[
            pltpu.VMEM((ROWS_PER_W, 2, K), jnp.int32),
            pltpu.VMEM((K,), jnp.float32),
            pltpu.VMEM((STRIPE,), jnp.float32),
            pltpu.VMEM_SHARED((NPAD,), jnp.float32),
            pltpu.SemaphoreType.DMA,
        ],
    )
    return kern(edges4d)


PHASE = 40


def _msg_body(edges_hbm, hs_hbm, out_hbm,
              idx_v, rows0, rows1, acc_sh, sem0, sem1):
    c = lax.axis_index("c")
    s = lax.axis_index("s")
    w = s * NC + c

    def zr(t, carry):
        rows0[t >> 3, pl.ds((t & 7) * 16, 16)] = jnp.zeros((16,), jnp.float32)
        return carry

    lax.fori_loop(0, K * (DIM // 16), zr, 0)
    for t in range(STRIPE // K):
        pltpu.sync_copy(rows0, acc_sh.at[pl.ds(s * STRIPE + t * K, K)])
    plsc.subcore_barrier()

    pltpu.sync_copy(edges_hbm.at[w], idx_v)

    def step(j, carry):
        pltpu.async_copy(hs_hbm.at[idx_v.at[j, 0]], rows0, sem0).wait()
        pltpu.sync_copy(rows0, acc_sh.at[idx_v.at[j, 1]], add=True)
        return carry

    lax.fori_loop(0, ROWS_PER_W, step, 0)
    plsc.subcore_barrier()
    pltpu.sync_copy(acc_sh.at[pl.ds(s * STRIPE, STRIPE)],
                    out_hbm.at[c, pl.ds(s * STRIPE, STRIPE)])


def _sc_edges(edges4d, hs):
    kern = pl.kernel(
        _msg_body,
        out_type=jax.ShapeDtypeStruct((NC, NPAD, DIM), jnp.float32),
        mesh=_mesh(),
        scratch_types=[
            pltpu.VMEM((ROWS_PER_W, 2, K), jnp.int32),
            pltpu.VMEM((K, DIM), jnp.float32),
            pltpu.VMEM((K, DIM), jnp.float32),
            pltpu.VMEM_SHARED((NPAD, DIM), jnp.float32),
            pltpu.SemaphoreType.DMA,
            pltpu.SemaphoreType.DMA,
        ],
    )
    return kern(edges4d, hs)


def _prep_body(x_ref, deg_ref, w_ref, g_ref, bt_ref, hs_ref):
    x = x_ref[...]
    mu = jnp.mean(x, axis=-1, keepdims=True)
    var = jnp.mean(x * x, axis=-1, keepdims=True) - mu * mu
    xr = (x - mu) * lax.rsqrt(var + 1e-5) * g_ref[...] + bt_ref[...]
    h = jnp.dot(xr, w_ref[...], preferred_element_type=jnp.float32)
    deg = jnp.sum(deg_ref[...], axis=-1, keepdims=True) + 1.0
    hs_ref[...] = h * lax.rsqrt(deg)


def _tc_prep(x, deg2, W, gamma, beta):
    B = 1000
    return pl.pallas_call(
        _prep_body,
        grid=(N // B,),
        in_specs=[
            pl.BlockSpec((B, DIM), lambda i: (i, 0)),
            pl.BlockSpec((B, 2), lambda i: (i, 0)),
            pl.BlockSpec((DIM, DIM), lambda i: (0, 0)),
            pl.BlockSpec((DIM,), lambda i: (0,)),
            pl.BlockSpec((DIM,), lambda i: (0,)),
        ],
        out_specs=pl.BlockSpec((B, DIM), lambda i: (i, 0)),
        out_shape=jax.ShapeDtypeStruct((N, DIM), jnp.float32),
    )(x, deg2, W, gamma, beta)


def _final_body(x_ref, hs_ref, acc_ref, deg_ref, b_ref, o_ref):
    q = acc_ref[0] + acc_ref[1] + hs_ref[...]
    deg = jnp.sum(deg_ref[...], axis=-1, keepdims=True) + 1.0
    agg = q * lax.rsqrt(deg)
    o_ref[...] = x_ref[...] + jnp.maximum(agg + b_ref[...], 0.0)


def _tc_final(x, hs, acc, deg2, b):
    B = 1000
    return pl.pallas_call(
        _final_body,
        grid=(N // B,),
        in_specs=[
            pl.BlockSpec((B, DIM), lambda i: (i, 0)),
            pl.BlockSpec((B, DIM), lambda i: (i, 0)),
            pl.BlockSpec((NC, B, DIM), lambda i: (0, i, 0)),
            pl.BlockSpec((B, 2), lambda i: (i, 0)),
            pl.BlockSpec((DIM,), lambda i: (0,)),
        ],
        out_specs=pl.BlockSpec((B, DIM), lambda i: (i, 0)),
        out_shape=jax.ShapeDtypeStruct((N, DIM), jnp.float32),
    )(x, hs, acc, deg2, b)


def kernel(x, edge_index, A_norm, edge_attr, W, b, gamma, beta):
    pad = EPAD - E
    src3d = jnp.concatenate(
        [edge_index[0], jnp.zeros((pad,), jnp.int32)]).reshape(
            NW, ROWS_PER_W, K)
    dst3d = jnp.concatenate(
        [edge_index[1], jnp.full((pad,), DUMP, jnp.int32)]).reshape(
            NW, ROWS_PER_W, K)
    edges4d = jnp.stack([src3d, dst3d], axis=2)
    deg_part = _sc_degree(edges4d)
    deg2 = deg_part.reshape(NC, NPAD)[:, :N].T
    hs = _tc_prep(x, deg2, W, gamma, beta)
    acc = _sc_edges(edges4d, hs)
    x_out = _tc_final(x, hs, acc, deg2, b)
    return (x_out, edge_attr)


# speedup vs baseline: 1.0726x; 1.0726x over previous
import jax
import jax.numpy as jnp
from jax import lax
from jax.experimental import pallas as pl
from jax.experimental.pallas import tpu as pltpu
from jax.experimental.pallas import tpu_sc as plsc

N = 10000
E = 320000
DIM = 128

NC = 2
NS = 16
NW = NC * NS
K = 128
ROWS_PER_W = 80
EPAD = NW * ROWS_PER_W * K
NPAD = 10240
STRIPE = NPAD // NS
DUMP = NPAD - 1


def _mesh():
    return plsc.VectorSubcoreMesh(core_axis_name="c", subcore_axis_name="s")


def _deg_body(edges_hbm, out_hbm, idx_v, ones_v, zero_v, deg_sh, sem):
    c = lax.axis_index("c")
    s = lax.axis_index("s")
    w = s * NC + c
    pltpu.sync_copy(edges_hbm.at[w], idx_v)
    for i in range(K // 16):
        ones_v[pl.ds(i * 16, 16)] = jnp.ones((16,), jnp.float32)

    def zb(t, carry):
        zero_v[pl.ds(t * 16, 16)] = jnp.zeros((16,), jnp.float32)
        return carry

    lax.fori_loop(0, STRIPE // 16, zb, 0)
    pltpu.sync_copy(zero_v, deg_sh.at[pl.ds(s * STRIPE, STRIPE)])
    plsc.subcore_barrier()

    def step(j, carry):
        pltpu.sync_copy(ones_v, deg_sh.at[idx_v.at[j, 1]], add=True)
        return carry

    lax.fori_loop(0, ROWS_PER_W, step, 0)
    plsc.subcore_barrier()
    pltpu.sync_copy(deg_sh.at[pl.ds(s * STRIPE, STRIPE)],
                    out_hbm.at[c, 0, pl.ds(s * STRIPE, STRIPE)])


def _sc_degree(edges4d):
    kern = pl.kernel(
        _deg_body,
        out_type=jax.ShapeDtypeStruct((NC, 1, NPAD), jnp.float32),
        mesh=_mesh(),
        scratch_types=[
            pltpu.VMEM((ROWS_PER_W, 2, K), jnp.int32),
            pltpu.VMEM((K,), jnp.float32),
            pltpu.VMEM((STRIPE,), jnp.float32),
            pltpu.VMEM_SHARED((NPAD,), jnp.float32),
            pltpu.SemaphoreType.DMA,
        ],
    )
    return kern(edges4d)


NBUF = 2
PHASE = 40


def _msg_body(edges_hbm, hs_hbm, out_hbm, idx_v, rows_v, acc_sh, *sems):
    c = lax.axis_index("c")
    s = lax.axis_index("s")
    w = s * NC + c

    def slot(b):
        return rows_v.at[b]

    def zr(t, carry):
        rows_v[0, t >> 3, pl.ds((t & 7) * 16, 16)] = jnp.zeros((16,),
                                                               jnp.float32)
        return carry

    lax.fori_loop(0, K * (DIM // 16), zr, 0)
    for t in range(STRIPE // K):
        pltpu.sync_copy(slot(0), acc_sh.at[pl.ds(s * STRIPE + t * K, K)])
    plsc.subcore_barrier()

    sem = sems[0]

    def batch(g2, carry):
        g = g2 * NBUF
        for b in range(NBUF):
            pltpu.async_copy(hs_hbm.at[idx_v.at[g + b, 0]], slot(b), sem)
        for b in range(NBUF):
            pltpu.make_async_copy(hs_hbm.at[pl.ds(0, K)], slot(b), sem).wait()
            pltpu.sync_copy(slot(b), acc_sh.at[idx_v.at[g + b, 1]], add=True)
        return carry

    for phase in range(ROWS_PER_W // PHASE):
        pltpu.sync_copy(edges_hbm.at[w, pl.ds(phase * PHASE, PHASE)], idx_v)
        lax.fori_loop(0, PHASE // NBUF, batch, 0)

    plsc.subcore_barrier()
    pltpu.sync_copy(acc_sh.at[pl.ds(s * STRIPE, STRIPE)],
                    out_hbm.at[c, pl.ds(s * STRIPE, STRIPE)])


def _sc_edges(edges4d, hs):
    kern = pl.kernel(
        _msg_body,
        out_type=jax.ShapeDtypeStruct((NC, NPAD, DIM), jnp.float32),
        mesh=_mesh(),
        scratch_types=(
            [pltpu.VMEM((PHASE, 2, K), jnp.int32)]
            + [pltpu.VMEM((NBUF, K, DIM), jnp.float32)]
            + [pltpu.VMEM_SHARED((NPAD, DIM), jnp.float32)]
            + [pltpu.SemaphoreType.DMA]
        ),
    )
    return kern(edges4d, hs)


def _prep_body(x_ref, deg_ref, w_ref, g_ref, bt_ref, hs_ref):
    x = x_ref[...]
    mu = jnp.mean(x, axis=-1, keepdims=True)
    var = jnp.mean(x * x, axis=-1, keepdims=True) - mu * mu
    xr = (x - mu) * lax.rsqrt(var + 1e-5) * g_ref[...] + bt_ref[...]
    h = jnp.dot(xr, w_ref[...], preferred_element_type=jnp.float32)
    deg = jnp.sum(deg_ref[...], axis=-1, keepdims=True) + 1.0
    hs_ref[...] = h * lax.rsqrt(deg)


def _tc_prep(x, deg2, W, gamma, beta):
    B = 1000
    return pl.pallas_call(
        _prep_body,
        grid=(N // B,),
        in_specs=[
            pl.BlockSpec((B, DIM), lambda i: (i, 0)),
            pl.BlockSpec((B, 2), lambda i: (i, 0)),
            pl.BlockSpec((DIM, DIM), lambda i: (0, 0)),
            pl.BlockSpec((DIM,), lambda i: (0,)),
            pl.BlockSpec((DIM,), lambda i: (0,)),
        ],
        out_specs=pl.BlockSpec((B, DIM), lambda i: (i, 0)),
        out_shape=jax.ShapeDtypeStruct((N, DIM), jnp.float32),
    )(x, deg2, W, gamma, beta)


def _final_body(x_ref, hs_ref, acc_ref, deg_ref, b_ref, o_ref):
    q = acc_ref[0] + acc_ref[1] + hs_ref[...]
    deg = jnp.sum(deg_ref[...], axis=-1, keepdims=True) + 1.0
    agg = q * lax.rsqrt(deg)
    o_ref[...] = x_ref[...] + jnp.maximum(agg + b_ref[...], 0.0)


def _tc_final(x, hs, acc, deg2, b):
    B = 1000
    return pl.pallas_call(
        _final_body,
        grid=(N // B,),
        in_specs=[
            pl.BlockSpec((B, DIM), lambda i: (i, 0)),
            pl.BlockSpec((B, DIM), lambda i: (i, 0)),
            pl.BlockSpec((NC, B, DIM), lambda i: (0, i, 0)),
            pl.BlockSpec((B, 2), lambda i: (i, 0)),
            pl.BlockSpec((DIM,), lambda i: (0,)),
        ],
        out_specs=pl.BlockSpec((B, DIM), lambda i: (i, 0)),
        out_shape=jax.ShapeDtypeStruct((N, DIM), jnp.float32),
    )(x, hs, acc, deg2, b)


def kernel(x, edge_index, A_norm, edge_attr, W, b, gamma, beta):
    pad = EPAD - E
    src3d = jnp.concatenate(
        [edge_index[0], jnp.zeros((pad,), jnp.int32)]).reshape(
            NW, ROWS_PER_W, K)
    dst3d = jnp.concatenate(
        [edge_index[1], jnp.full((pad,), DUMP, jnp.int32)]).reshape(
            NW, ROWS_PER_W, K)
    edges4d = jnp.stack([src3d, dst3d], axis=2)
    deg_part = _sc_degree(edges4d)
    deg2 = deg_part.reshape(NC, NPAD)[:, :N].T
    hs = _tc_prep(x, deg2, W, gamma, beta)
    acc = _sc_edges(edges4d, hs)
    x_out = _tc_final(x, hs, acc, deg2, b)
    return (x_out, edge_attr)


# speedup vs baseline: 2.1318x; 1.9875x over previous
import jax
import jax.numpy as jnp
from jax import lax
from jax.experimental import pallas as pl
from jax.experimental.pallas import tpu as pltpu
from jax.experimental.pallas import tpu_sc as plsc

N = 10000
E = 320000
DIM = 128

NC = 2
NS = 16
NW = NC * NS
K = 80
ROWS_PER_W = 125
EPAD = NW * ROWS_PER_W * K
NPAD = 10240
STRIPE = NPAD // NS
DUMP = NPAD - 1


def _mesh():
    return plsc.VectorSubcoreMesh(core_axis_name="c", subcore_axis_name="s")


def _deg_body(edges_hbm, out_hbm, idx_v, ones_v, zero_v, deg_sh, sem):
    c = lax.axis_index("c")
    s = lax.axis_index("s")
    w = s * NC + c
    pltpu.sync_copy(edges_hbm.at[w], idx_v)
    for i in range(K // 16):
        ones_v[pl.ds(i * 16, 16)] = jnp.ones((16,), jnp.float32)

    def zb(t, carry):
        zero_v[pl.ds(t * 16, 16)] = jnp.zeros((16,), jnp.float32)
        return carry

    lax.fori_loop(0, STRIPE // 16, zb, 0)
    pltpu.sync_copy(zero_v, deg_sh.at[pl.ds(s * STRIPE, STRIPE)])
    plsc.subcore_barrier()

    def step(j, carry):
        pltpu.sync_copy(ones_v, deg_sh.at[idx_v.at[j, 1]], add=True)
        return carry

    lax.fori_loop(0, ROWS_PER_W, step, 0)
    plsc.subcore_barrier()
    pltpu.sync_copy(deg_sh.at[pl.ds(s * STRIPE, STRIPE)],
                    out_hbm.at[c, 0, pl.ds(s * STRIPE, STRIPE)])


def _sc_degree(edges4d):
    kern = pl.kernel(
        _deg_body,
        out_type=jax.ShapeDtypeStruct((NC, 1, NPAD), jnp.float32),
        mesh=_mesh(),
        scratch_types=[
            pltpu.VMEM((ROWS_PER_W, 2, K), jnp.int32),
            pltpu.VMEM((K,), jnp.float32),
            pltpu.VMEM((STRIPE,), jnp.float32),
            pltpu.VMEM_SHARED((NPAD,), jnp.float32),
            pltpu.SemaphoreType.DMA,
        ],
    )
    return kern(edges4d)


NBUF = 2
PHASE = 25


def _msg_body(edges_hbm, hs_hbm, out_hbm, idx_v, rows_v, acc_sh, *sems):
    c = lax.axis_index("c")
    s = lax.axis_index("s")
    w = s * NC + c

    def slot(b):
        return rows_v.at[b]

    def zr(t, carry):
        rows_v[0, t >> 3, pl.ds((t & 7) * 16, 16)] = jnp.zeros((16,),
                                                               jnp.float32)
        return carry

    lax.fori_loop(0, K * (DIM // 16), zr, 0)
    for t in range(STRIPE // K):
        pltpu.sync_copy(slot(0), acc_sh.at[pl.ds(s * STRIPE + t * K, K)])
    plsc.subcore_barrier()

    sem = sems[0]

    def batch(g2, carry):
        g = g2 * NBUF
        for b in range(NBUF):
            pltpu.async_copy(hs_hbm.at[idx_v.at[g + b, 0]], slot(b), sem)
        for b in range(NBUF):
            pltpu.make_async_copy(hs_hbm.at[pl.ds(0, K)], slot(b), sem).wait()
            pltpu.sync_copy(slot(b), acc_sh.at[idx_v.at[g + b, 1]], add=True)
        return carry

    for phase in range(ROWS_PER_W // PHASE):
        pltpu.sync_copy(edges_hbm.at[w, pl.ds(phase * PHASE, PHASE)], idx_v)
        lax.fori_loop(0, PHASE // NBUF, batch, 0)
        for r in range(PHASE % NBUF):
            j = PHASE - (PHASE % NBUF) + r
            pltpu.async_copy(hs_hbm.at[idx_v.at[j, 0]], slot(r), sem)
            pltpu.make_async_copy(hs_hbm.at[pl.ds(0, K)], slot(r), sem).wait()
            pltpu.sync_copy(slot(r), acc_sh.at[idx_v.at[j, 1]], add=True)

    plsc.subcore_barrier()
    pltpu.sync_copy(acc_sh.at[pl.ds(s * STRIPE, STRIPE)],
                    out_hbm.at[c, pl.ds(s * STRIPE, STRIPE)])


def _sc_edges(edges4d, hs):
    kern = pl.kernel(
        _msg_body,
        out_type=jax.ShapeDtypeStruct((NC, NPAD, DIM), jnp.float32),
        mesh=_mesh(),
        scratch_types=(
            [pltpu.VMEM((PHASE, 2, K), jnp.int32)]
            + [pltpu.VMEM((NBUF, K, DIM), jnp.float32)]
            + [pltpu.VMEM_SHARED((NPAD, DIM), jnp.float32)]
            + [pltpu.SemaphoreType.DMA]
        ),
    )
    return kern(edges4d, hs)


def _prep_body(x_ref, deg_ref, w_ref, g_ref, bt_ref, hs_ref):
    x = x_ref[...]
    mu = jnp.mean(x, axis=-1, keepdims=True)
    var = jnp.mean(x * x, axis=-1, keepdims=True) - mu * mu
    xr = (x - mu) * lax.rsqrt(var + 1e-5) * g_ref[...] + bt_ref[...]
    h = jnp.dot(xr, w_ref[...], preferred_element_type=jnp.float32)
    deg = jnp.sum(deg_ref[...], axis=-1, keepdims=True) + 1.0
    hs_ref[...] = h * lax.rsqrt(deg)


def _tc_prep(x, deg2, W, gamma, beta):
    B = 1000
    return pl.pallas_call(
        _prep_body,
        grid=(N // B,),
        in_specs=[
            pl.BlockSpec((B, DIM), lambda i: (i, 0)),
            pl.BlockSpec((B, 2), lambda i: (i, 0)),
            pl.BlockSpec((DIM, DIM), lambda i: (0, 0)),
            pl.BlockSpec((DIM,), lambda i: (0,)),
            pl.BlockSpec((DIM,), lambda i: (0,)),
        ],
        out_specs=pl.BlockSpec((B, DIM), lambda i: (i, 0)),
        out_shape=jax.ShapeDtypeStruct((N, DIM), jnp.float32),
    )(x, deg2, W, gamma, beta)


def _final_body(x_ref, hs_ref, acc_ref, deg_ref, b_ref, o_ref):
    q = acc_ref[0] + acc_ref[1] + hs_ref[...]
    deg = jnp.sum(deg_ref[...], axis=-1, keepdims=True) + 1.0
    agg = q * lax.rsqrt(deg)
    o_ref[...] = x_ref[...] + jnp.maximum(agg + b_ref[...], 0.0)


def _tc_final(x, hs, acc, deg2, b):
    B = 1000
    return pl.pallas_call(
        _final_body,
        grid=(N // B,),
        in_specs=[
            pl.BlockSpec((B, DIM), lambda i: (i, 0)),
            pl.BlockSpec((B, DIM), lambda i: (i, 0)),
            pl.BlockSpec((NC, B, DIM), lambda i: (0, i, 0)),
            pl.BlockSpec((B, 2), lambda i: (i, 0)),
            pl.BlockSpec((DIM,), lambda i: (0,)),
        ],
        out_specs=pl.BlockSpec((B, DIM), lambda i: (i, 0)),
        out_shape=jax.ShapeDtypeStruct((N, DIM), jnp.float32),
    )(x, hs, acc, deg2, b)


def kernel(x, edge_index, A_norm, edge_attr, W, b, gamma, beta):
    pad = EPAD - E
    src3d = jnp.concatenate(
        [edge_index[0], jnp.zeros((pad,), jnp.int32)]).reshape(
            NW, ROWS_PER_W, K)
    dst3d = jnp.concatenate(
        [edge_index[1], jnp.full((pad,), DUMP, jnp.int32)]).reshape(
            NW, ROWS_PER_W, K)
    edges4d = jnp.stack([src3d, dst3d], axis=2)
    deg_part = _sc_degree(edges4d)
    deg2 = deg_part.reshape(NC, NPAD)[:, :N].T
    hs = _tc_prep(x, deg2, W, gamma, beta)
    acc = _sc_edges(edges4d, hs)
    x_out = _tc_final(x, hs, acc, deg2, b)
    return (x_out, edge_attr)


# speedup vs baseline: 2.2859x; 1.0723x over previous
import jax
import jax.numpy as jnp
from jax import lax
from jax.experimental import pallas as pl
from jax.experimental.pallas import tpu as pltpu
from jax.experimental.pallas import tpu_sc as plsc

N = 10000
E = 320000
DIM = 128

NC = 2
NS = 16
NW = NC * NS
K = 80
ROWS_PER_W = 125
EPAD = NW * ROWS_PER_W * K
NPAD = 10240
STRIPE = NPAD // NS
DUMP = NPAD - 1


def _mesh():
    return plsc.VectorSubcoreMesh(core_axis_name="c", subcore_axis_name="s")


def _deg_body(edges_hbm, out_hbm, idx_v, ones_v, zero_v, deg_sh, sem):
    c = lax.axis_index("c")
    s = lax.axis_index("s")
    w = s * NC + c
    pltpu.sync_copy(edges_hbm.at[w], idx_v)
    for i in range(K // 16):
        ones_v[pl.ds(i * 16, 16)] = jnp.ones((16,), jnp.float32)

    def zb(t, carry):
        zero_v[pl.ds(t * 16, 16)] = jnp.zeros((16,), jnp.float32)
        return carry

    lax.fori_loop(0, STRIPE // 16, zb, 0)
    pltpu.sync_copy(zero_v, deg_sh.at[pl.ds(s * STRIPE, STRIPE)])
    plsc.subcore_barrier()

    def step(j, carry):
        pltpu.sync_copy(ones_v, deg_sh.at[idx_v.at[j, 1]], add=True)
        return carry

    lax.fori_loop(0, ROWS_PER_W, step, 0)
    plsc.subcore_barrier()
    pltpu.sync_copy(deg_sh.at[pl.ds(s * STRIPE, STRIPE)],
                    out_hbm.at[c, 0, pl.ds(s * STRIPE, STRIPE)])


def _sc_degree(edges4d):
    kern = pl.kernel(
        _deg_body,
        out_type=jax.ShapeDtypeStruct((NC, 1, NPAD), jnp.float32),
        mesh=_mesh(),
        scratch_types=[
            pltpu.VMEM((ROWS_PER_W, 2, K), jnp.int32),
            pltpu.VMEM((K,), jnp.float32),
            pltpu.VMEM((STRIPE,), jnp.float32),
            pltpu.VMEM_SHARED((NPAD,), jnp.float32),
            pltpu.SemaphoreType.DMA,
        ],
    )
    return kern(edges4d)


NBUF = 3
PHASE = 25


def _msg_body(edges_hbm, hs_hbm, out_hbm, idx_v, rows_v, acc_sh, *sems):
    c = lax.axis_index("c")
    s = lax.axis_index("s")
    w = s * NC + c

    def slot(b):
        return rows_v.at[b]

    def zr(t, carry):
        rows_v[0, t >> 3, pl.ds((t & 7) * 16, 16)] = jnp.zeros((16,),
                                                               jnp.float32)
        return carry

    lax.fori_loop(0, K * (DIM // 16), zr, 0)
    for t in range(STRIPE // K):
        pltpu.sync_copy(slot(0), acc_sh.at[pl.ds(s * STRIPE + t * K, K)])
    plsc.subcore_barrier()

    sem = sems[0]

    def batch(g2, carry):
        g = g2 * NBUF
        for b in range(NBUF):
            pltpu.async_copy(hs_hbm.at[idx_v.at[g + b, 0]], slot(b), sem)
        for b in range(NBUF):
            pltpu.make_async_copy(hs_hbm.at[pl.ds(0, K)], slot(b), sem).wait()
            pltpu.sync_copy(slot(b), acc_sh.at[idx_v.at[g + b, 1]], add=True)
        return carry

    for phase in range(ROWS_PER_W // PHASE):
        pltpu.sync_copy(edges_hbm.at[w, pl.ds(phase * PHASE, PHASE)], idx_v)
        lax.fori_loop(0, PHASE // NBUF, batch, 0)
        for r in range(PHASE % NBUF):
            j = PHASE - (PHASE % NBUF) + r
            pltpu.async_copy(hs_hbm.at[idx_v.at[j, 0]], slot(r), sem)
            pltpu.make_async_copy(hs_hbm.at[pl.ds(0, K)], slot(r), sem).wait()
            pltpu.sync_copy(slot(r), acc_sh.at[idx_v.at[j, 1]], add=True)

    plsc.subcore_barrier()
    pltpu.sync_copy(acc_sh.at[pl.ds(s * STRIPE, STRIPE)],
                    out_hbm.at[c, pl.ds(s * STRIPE, STRIPE)])


def _sc_edges(edges4d, hs):
    kern = pl.kernel(
        _msg_body,
        out_type=jax.ShapeDtypeStruct((NC, NPAD, DIM), jnp.float32),
        mesh=_mesh(),
        scratch_types=(
            [pltpu.VMEM((PHASE, 2, K), jnp.int32)]
            + [pltpu.VMEM((NBUF, K, DIM), jnp.float32)]
            + [pltpu.VMEM_SHARED((NPAD, DIM), jnp.float32)]
            + [pltpu.SemaphoreType.DMA]
        ),
    )
    return kern(edges4d, hs)


def _prep_body(x_ref, deg_ref, w_ref, g_ref, bt_ref, hs_ref):
    x = x_ref[...]
    mu = jnp.mean(x, axis=-1, keepdims=True)
    var = jnp.mean(x * x, axis=-1, keepdims=True) - mu * mu
    xr = (x - mu) * lax.rsqrt(var + 1e-5) * g_ref[...] + bt_ref[...]
    h = jnp.dot(xr, w_ref[...], preferred_element_type=jnp.float32)
    deg = jnp.sum(deg_ref[...], axis=-1, keepdims=True) + 1.0
    hs_ref[...] = h * lax.rsqrt(deg)


def _tc_prep(x, deg2, W, gamma, beta):
    B = 1000
    return pl.pallas_call(
        _prep_body,
        grid=(N // B,),
        in_specs=[
            pl.BlockSpec((B, DIM), lambda i: (i, 0)),
            pl.BlockSpec((B, 2), lambda i: (i, 0)),
            pl.BlockSpec((DIM, DIM), lambda i: (0, 0)),
            pl.BlockSpec((DIM,), lambda i: (0,)),
            pl.BlockSpec((DIM,), lambda i: (0,)),
        ],
        out_specs=pl.BlockSpec((B, DIM), lambda i: (i, 0)),
        out_shape=jax.ShapeDtypeStruct((N, DIM), jnp.float32),
    )(x, deg2, W, gamma, beta)


def _final_body(x_ref, hs_ref, acc_ref, deg_ref, b_ref, o_ref):
    q = acc_ref[0] + acc_ref[1] + hs_ref[...]
    deg = jnp.sum(deg_ref[...], axis=-1, keepdims=True) + 1.0
    agg = q * lax.rsqrt(deg)
    o_ref[...] = x_ref[...] + jnp.maximum(agg + b_ref[...], 0.0)


def _tc_final(x, hs, acc, deg2, b):
    B = 1000
    return pl.pallas_call(
        _final_body,
        grid=(N // B,),
        in_specs=[
            pl.BlockSpec((B, DIM), lambda i: (i, 0)),
            pl.BlockSpec((B, DIM), lambda i: (i, 0)),
            pl.BlockSpec((NC, B, DIM), lambda i: (0, i, 0)),
            pl.BlockSpec((B, 2), lambda i: (i, 0)),
            pl.BlockSpec((DIM,), lambda i: (0,)),
        ],
        out_specs=pl.BlockSpec((B, DIM), lambda i: (i, 0)),
        out_shape=jax.ShapeDtypeStruct((N, DIM), jnp.float32),
    )(x, hs, acc, deg2, b)


def kernel(x, edge_index, A_norm, edge_attr, W, b, gamma, beta):
    pad = EPAD - E
    src3d = jnp.concatenate(
        [edge_index[0], jnp.zeros((pad,), jnp.int32)]).reshape(
            NW, ROWS_PER_W, K)
    dst3d = jnp.concatenate(
        [edge_index[1], jnp.full((pad,), DUMP, jnp.int32)]).reshape(
            NW, ROWS_PER_W, K)
    edges4d = jnp.stack([src3d, dst3d], axis=2)
    deg_part = _sc_degree(edges4d)
    deg2 = deg_part.reshape(NC, NPAD)[:, :N].T
    hs = _tc_prep(x, deg2, W, gamma, beta)
    acc = _sc_edges(edges4d, hs)
    x_out = _tc_final(x, hs, acc, deg2, b)
    return (x_out, edge_attr)


# speedup vs baseline: 2.3473x; 1.0269x over previous
import jax
import jax.numpy as jnp
from jax import lax
from jax.experimental import pallas as pl
from jax.experimental.pallas import tpu as pltpu
from jax.experimental.pallas import tpu_sc as plsc

N = 10000
E = 320000
DIM = 128

NC = 2
NS = 16
NW = NC * NS
K = 80
ROWS_PER_W = 125
EPAD = NW * ROWS_PER_W * K
NPAD = 10240
STRIPE = NPAD // NS
DUMP = NPAD - 1


def _mesh():
    return plsc.VectorSubcoreMesh(core_axis_name="c", subcore_axis_name="s")


def _deg_body(edges_hbm, out_hbm, idx_v, ones_v, zero_v, deg_sh, sem):
    c = lax.axis_index("c")
    s = lax.axis_index("s")
    w = s * NC + c
    pltpu.sync_copy(edges_hbm.at[w], idx_v)
    for i in range(K // 16):
        ones_v[pl.ds(i * 16, 16)] = jnp.ones((16,), jnp.float32)

    def zb(t, carry):
        zero_v[pl.ds(t * 16, 16)] = jnp.zeros((16,), jnp.float32)
        return carry

    lax.fori_loop(0, STRIPE // 16, zb, 0)
    pltpu.sync_copy(zero_v, deg_sh.at[pl.ds(s * STRIPE, STRIPE)])
    plsc.subcore_barrier()

    def step(j, carry):
        pltpu.sync_copy(ones_v, deg_sh.at[idx_v.at[j, 1]], add=True)
        return carry

    lax.fori_loop(0, ROWS_PER_W, step, 0)
    plsc.subcore_barrier()
    pltpu.sync_copy(deg_sh.at[pl.ds(s * STRIPE, STRIPE)],
                    out_hbm.at[c, 0, pl.ds(s * STRIPE, STRIPE)])


def _sc_degree(edges4d):
    kern = pl.kernel(
        _deg_body,
        out_type=jax.ShapeDtypeStruct((NC, 1, NPAD), jnp.float32),
        mesh=_mesh(),
        scratch_types=[
            pltpu.VMEM((ROWS_PER_W, 2, K), jnp.int32),
            pltpu.VMEM((K,), jnp.float32),
            pltpu.VMEM((STRIPE,), jnp.float32),
            pltpu.VMEM_SHARED((NPAD,), jnp.float32),
            pltpu.SemaphoreType.DMA,
        ],
    )
    return kern(edges4d)


NBUF = 4
PHASE = 25


def _msg_body(edges_hbm, hs_hbm, out_hbm, idx_v, rows_v, acc_sh, *sems):
    c = lax.axis_index("c")
    s = lax.axis_index("s")
    w = s * NC + c

    def slot(b):
        return rows_v.at[b]

    def zr(t, carry):
        rows_v[0, t >> 3, pl.ds((t & 7) * 16, 16)] = jnp.zeros((16,),
                                                               jnp.float32)
        return carry

    lax.fori_loop(0, K * (DIM // 16), zr, 0)
    for t in range(STRIPE // K):
        pltpu.sync_copy(slot(0), acc_sh.at[pl.ds(s * STRIPE + t * K, K)])
    plsc.subcore_barrier()

    sem = sems[0]

    def batch(g2, carry):
        g = g2 * NBUF
        for b in range(NBUF):
            pltpu.async_copy(hs_hbm.at[idx_v.at[g + b, 0]], slot(b), sem)
        for b in range(NBUF):
            pltpu.make_async_copy(hs_hbm.at[pl.ds(0, K)], slot(b), sem).wait()
            pltpu.sync_copy(slot(b), acc_sh.at[idx_v.at[g + b, 1]], add=True)
        return carry

    for phase in range(ROWS_PER_W // PHASE):
        pltpu.sync_copy(edges_hbm.at[w, pl.ds(phase * PHASE, PHASE)], idx_v)
        lax.fori_loop(0, PHASE // NBUF, batch, 0)
        for r in range(PHASE % NBUF):
            j = PHASE - (PHASE % NBUF) + r
            pltpu.async_copy(hs_hbm.at[idx_v.at[j, 0]], slot(r), sem)
            pltpu.make_async_copy(hs_hbm.at[pl.ds(0, K)], slot(r), sem).wait()
            pltpu.sync_copy(slot(r), acc_sh.at[idx_v.at[j, 1]], add=True)

    plsc.subcore_barrier()
    pltpu.sync_copy(acc_sh.at[pl.ds(s * STRIPE, STRIPE)],
                    out_hbm.at[c, pl.ds(s * STRIPE, STRIPE)])


def _sc_edges(edges4d, hs):
    kern = pl.kernel(
        _msg_body,
        out_type=jax.ShapeDtypeStruct((NC, NPAD, DIM), jnp.float32),
        mesh=_mesh(),
        scratch_types=(
            [pltpu.VMEM((PHASE, 2, K), jnp.int32)]
            + [pltpu.VMEM((NBUF, K, DIM), jnp.float32)]
            + [pltpu.VMEM_SHARED((NPAD, DIM), jnp.float32)]
            + [pltpu.SemaphoreType.DMA]
        ),
    )
    return kern(edges4d, hs)


def _prep_body(x_ref, deg_ref, w_ref, g_ref, bt_ref, hs_ref):
    x = x_ref[...]
    mu = jnp.mean(x, axis=-1, keepdims=True)
    var = jnp.mean(x * x, axis=-1, keepdims=True) - mu * mu
    xr = (x - mu) * lax.rsqrt(var + 1e-5) * g_ref[...] + bt_ref[...]
    h = jnp.dot(xr, w_ref[...], preferred_element_type=jnp.float32)
    deg = jnp.sum(deg_ref[...], axis=-1, keepdims=True) + 1.0
    hs_ref[...] = h * lax.rsqrt(deg)


def _tc_prep(x, deg2, W, gamma, beta):
    B = 1000
    return pl.pallas_call(
        _prep_body,
        grid=(N // B,),
        in_specs=[
            pl.BlockSpec((B, DIM), lambda i: (i, 0)),
            pl.BlockSpec((B, 2), lambda i: (i, 0)),
            pl.BlockSpec((DIM, DIM), lambda i: (0, 0)),
            pl.BlockSpec((DIM,), lambda i: (0,)),
            pl.BlockSpec((DIM,), lambda i: (0,)),
        ],
        out_specs=pl.BlockSpec((B, DIM), lambda i: (i, 0)),
        out_shape=jax.ShapeDtypeStruct((N, DIM), jnp.float32),
    )(x, deg2, W, gamma, beta)


def _final_body(x_ref, hs_ref, acc_ref, deg_ref, b_ref, o_ref):
    q = acc_ref[0] + acc_ref[1] + hs_ref[...]
    deg = jnp.sum(deg_ref[...], axis=-1, keepdims=True) + 1.0
    agg = q * lax.rsqrt(deg)
    o_ref[...] = x_ref[...] + jnp.maximum(agg + b_ref[...], 0.0)


def _tc_final(x, hs, acc, deg2, b):
    B = 1000
    return pl.pallas_call(
        _final_body,
        grid=(N // B,),
        in_specs=[
            pl.BlockSpec((B, DIM), lambda i: (i, 0)),
            pl.BlockSpec((B, DIM), lambda i: (i, 0)),
            pl.BlockSpec((NC, B, DIM), lambda i: (0, i, 0)),
            pl.BlockSpec((B, 2), lambda i: (i, 0)),
            pl.BlockSpec((DIM,), lambda i: (0,)),
        ],
        out_specs=pl.BlockSpec((B, DIM), lambda i: (i, 0)),
        out_shape=jax.ShapeDtypeStruct((N, DIM), jnp.float32),
    )(x, hs, acc, deg2, b)


def kernel(x, edge_index, A_norm, edge_attr, W, b, gamma, beta):
    pad = EPAD - E
    src3d = jnp.concatenate(
        [edge_index[0], jnp.zeros((pad,), jnp.int32)]).reshape(
            NW, ROWS_PER_W, K)
    dst3d = jnp.concatenate(
        [edge_index[1], jnp.full((pad,), DUMP, jnp.int32)]).reshape(
            NW, ROWS_PER_W, K)
    edges4d = jnp.stack([src3d, dst3d], axis=2)
    deg_part = _sc_degree(edges4d)
    deg2 = deg_part.reshape(NC, NPAD)[:, :N].T
    hs = _tc_prep(x, deg2, W, gamma, beta)
    acc = _sc_edges(edges4d, hs)
    x_out = _tc_final(x, hs, acc, deg2, b)
    return (x_out, edge_attr)


# speedup vs baseline: 2.3524x; 1.0021x over previous
import jax
import jax.numpy as jnp
from jax import lax
from jax.experimental import pallas as pl
from jax.experimental.pallas import tpu as pltpu
from jax.experimental.pallas import tpu_sc as plsc

N = 10000
E = 320000
DIM = 128

NC = 2
NS = 16
NW = NC * NS
K = 80
ROWS_PER_W = 125
EPAD = NW * ROWS_PER_W * K
NPAD = 10240
STRIPE = NPAD // NS
DUMP = NPAD - 1


def _mesh():
    return plsc.VectorSubcoreMesh(core_axis_name="c", subcore_axis_name="s")


def _deg_body(edges_hbm, out_hbm, idx_v, ones_v, zero_v, deg_sh, sem):
    c = lax.axis_index("c")
    s = lax.axis_index("s")
    w = s * NC + c
    pltpu.sync_copy(edges_hbm.at[w], idx_v)
    for i in range(K // 16):
        ones_v[pl.ds(i * 16, 16)] = jnp.ones((16,), jnp.float32)

    def zb(t, carry):
        zero_v[pl.ds(t * 16, 16)] = jnp.zeros((16,), jnp.float32)
        return carry

    lax.fori_loop(0, STRIPE // 16, zb, 0)
    pltpu.sync_copy(zero_v, deg_sh.at[pl.ds(s * STRIPE, STRIPE)])
    plsc.subcore_barrier()

    def step(j, carry):
        pltpu.sync_copy(ones_v, deg_sh.at[idx_v.at[j, 1]], add=True)
        return carry

    lax.fori_loop(0, ROWS_PER_W, step, 0)
    plsc.subcore_barrier()
    pltpu.sync_copy(deg_sh.at[pl.ds(s * STRIPE, STRIPE)],
                    out_hbm.at[c, 0, pl.ds(s * STRIPE, STRIPE)])


def _sc_degree(edges4d):
    kern = pl.kernel(
        _deg_body,
        out_type=jax.ShapeDtypeStruct((NC, 1, NPAD), jnp.float32),
        mesh=_mesh(),
        scratch_types=[
            pltpu.VMEM((ROWS_PER_W, 2, K), jnp.int32),
            pltpu.VMEM((K,), jnp.float32),
            pltpu.VMEM((STRIPE,), jnp.float32),
            pltpu.VMEM_SHARED((NPAD,), jnp.float32),
            pltpu.SemaphoreType.DMA,
        ],
    )
    return kern(edges4d)


NBUF = 4
PHASE = 25


def _msg_body(edges_hbm, hs_hbm, out_hbm, idx_v, rows_v, acc_sh, *sems):
    c = lax.axis_index("c")
    s = lax.axis_index("s")
    w = s * NC + c

    def slot(b):
        return rows_v.at[b]

    def zr(t, carry):
        rows_v[0, t >> 3, pl.ds((t & 7) * 16, 16)] = jnp.zeros((16,),
                                                               jnp.float32)
        return carry

    lax.fori_loop(0, K * (DIM // 16), zr, 0)
    for t in range(STRIPE // K):
        pltpu.sync_copy(slot(0), acc_sh.at[pl.ds(s * STRIPE + t * K, K)])
    plsc.subcore_barrier()

    sem = sems[0]

    def batch(g2, carry):
        g = g2 * NBUF
        for b in range(NBUF):
            pltpu.async_copy(hs_hbm.at[idx_v.at[g + b, 0]], slot(b), sem)
        for b in range(NBUF):
            pltpu.make_async_copy(hs_hbm.at[pl.ds(0, K)], slot(b), sem).wait()
            pltpu.sync_copy(slot(b), acc_sh.at[idx_v.at[g + b, 1]], add=True)
        return carry

    for phase in range(ROWS_PER_W // PHASE):
        pltpu.sync_copy(edges_hbm.at[w, pl.ds(phase * PHASE, PHASE)], idx_v)
        lax.fori_loop(0, PHASE // NBUF, batch, 0)
        for r in range(PHASE % NBUF):
            j = PHASE - (PHASE % NBUF) + r
            pltpu.async_copy(hs_hbm.at[idx_v.at[j, 0]], slot(r), sem)
            pltpu.make_async_copy(hs_hbm.at[pl.ds(0, K)], slot(r), sem).wait()
            pltpu.sync_copy(slot(r), acc_sh.at[idx_v.at[j, 1]], add=True)

    plsc.subcore_barrier()
    pltpu.sync_copy(acc_sh.at[pl.ds(s * STRIPE, STRIPE)],
                    out_hbm.at[c, pl.ds(s * STRIPE, STRIPE)])


def _sc_edges(edges4d, hs):
    kern = pl.kernel(
        _msg_body,
        out_type=jax.ShapeDtypeStruct((NC, NPAD, DIM), jnp.float32),
        mesh=_mesh(),
        scratch_types=(
            [pltpu.VMEM((PHASE, 2, K), jnp.int32)]
            + [pltpu.VMEM((NBUF, K, DIM), jnp.float32)]
            + [pltpu.VMEM_SHARED((NPAD, DIM), jnp.float32)]
            + [pltpu.SemaphoreType.DMA]
        ),
    )
    return kern(edges4d, hs)


def _prep1_body(x_ref, w_ref, g_ref, bt_ref, h_ref):
    x = x_ref[...]
    mu = jnp.mean(x, axis=-1, keepdims=True)
    var = jnp.mean(x * x, axis=-1, keepdims=True) - mu * mu
    xr = (x - mu) * lax.rsqrt(var + 1e-5) * g_ref[...] + bt_ref[...]
    h_ref[...] = jnp.dot(xr, w_ref[...], preferred_element_type=jnp.float32)


def _tc_prep1(x, W, gamma, beta):
    B = 1000
    return pl.pallas_call(
        _prep1_body,
        grid=(N // B,),
        in_specs=[
            pl.BlockSpec((B, DIM), lambda i: (i, 0)),
            pl.BlockSpec((DIM, DIM), lambda i: (0, 0)),
            pl.BlockSpec((DIM,), lambda i: (0,)),
            pl.BlockSpec((DIM,), lambda i: (0,)),
        ],
        out_specs=pl.BlockSpec((B, DIM), lambda i: (i, 0)),
        out_shape=jax.ShapeDtypeStruct((N, DIM), jnp.float32),
    )(x, W, gamma, beta)


def _prep2_body(h_ref, deg_ref, hs_ref):
    deg = jnp.sum(deg_ref[...], axis=-1, keepdims=True) + 1.0
    hs_ref[...] = h_ref[...] * lax.rsqrt(deg)


def _tc_prep2(h, deg2):
    B = 1000
    return pl.pallas_call(
        _prep2_body,
        grid=(N // B,),
        in_specs=[
            pl.BlockSpec((B, DIM), lambda i: (i, 0)),
            pl.BlockSpec((B, 2), lambda i: (i, 0)),
        ],
        out_specs=pl.BlockSpec((B, DIM), lambda i: (i, 0)),
        out_shape=jax.ShapeDtypeStruct((N, DIM), jnp.float32),
    )(h, deg2)


def _final_body(x_ref, hs_ref, acc_ref, deg_ref, b_ref, o_ref):
    q = acc_ref[0] + acc_ref[1] + hs_ref[...]
    deg = jnp.sum(deg_ref[...], axis=-1, keepdims=True) + 1.0
    agg = q * lax.rsqrt(deg)
    o_ref[...] = x_ref[...] + jnp.maximum(agg + b_ref[...], 0.0)


def _tc_final(x, hs, acc, deg2, b):
    B = 1000
    return pl.pallas_call(
        _final_body,
        grid=(N // B,),
        in_specs=[
            pl.BlockSpec((B, DIM), lambda i: (i, 0)),
            pl.BlockSpec((B, DIM), lambda i: (i, 0)),
            pl.BlockSpec((NC, B, DIM), lambda i: (0, i, 0)),
            pl.BlockSpec((B, 2), lambda i: (i, 0)),
            pl.BlockSpec((DIM,), lambda i: (0,)),
        ],
        out_specs=pl.BlockSpec((B, DIM), lambda i: (i, 0)),
        out_shape=jax.ShapeDtypeStruct((N, DIM), jnp.float32),
    )(x, hs, acc, deg2, b)


def kernel(x, edge_index, A_norm, edge_attr, W, b, gamma, beta):
    pad = EPAD - E
    src3d = jnp.concatenate(
        [edge_index[0], jnp.zeros((pad,), jnp.int32)]).reshape(
            NW, ROWS_PER_W, K)
    dst3d = jnp.concatenate(
        [edge_index[1], jnp.full((pad,), DUMP, jnp.int32)]).reshape(
            NW, ROWS_PER_W, K)
    edges4d = jnp.stack([src3d, dst3d], axis=2)
    h = _tc_prep1(x, W, gamma, beta)
    deg_part = _sc_degree(edges4d)
    deg2 = deg_part.reshape(NC, NPAD)[:, :N].T
    hs = _tc_prep2(h, deg2)
    acc = _sc_edges(edges4d, hs)
    x_out = _tc_final(x, hs, acc, deg2, b)
    return (x_out, edge_attr)
